# baseline (speedup 1.0000x reference)
import functools

import jax
import jax.numpy as jnp
from jax import lax
from jax.experimental import pallas as pl
from jax.experimental.pallas import tpu as pltpu
from jax.experimental.pallas import tpu_sc as plsc

_B, _S, _D, _V = 4, 8192, 1024, 8192
_NB = _B * _S
_NC, _NS = 2, 16
_NW = _NC * _NS
_BPW = _NB // _NW
_C = 16
_NCHUNK = _BPW // _C
_NSLOT = 4
_PREF = 3
_DRAIN = _NSLOT - _PREF

_mesh = plsc.VectorSubcoreMesh(core_axis_name="c", subcore_axis_name="s")


@functools.partial(
    pl.kernel,
    mesh=_mesh,
    out_type=jax.ShapeDtypeStruct((_NB, _D), jnp.float32),
    scratch_types=(
        [pltpu.VMEM((_NCHUNK, _C), jnp.int32)]
        + [pltpu.VMEM((_C, _D), jnp.float32)] * _NSLOT
        + [pltpu.SemaphoreType.DMA] * (2 * _NSLOT)
    ),
)
def _gather(idx_hbm, table_hbm, out_hbm, idx_v, *bufs_and_sems):
    bufs = bufs_and_sems[:_NSLOT]
    gsem = bufs_and_sems[_NSLOT:2 * _NSLOT]
    ssem = bufs_and_sems[2 * _NSLOT:]

    wid = lax.axis_index("s") * _NC + lax.axis_index("c")
    base = wid * _BPW
    pltpu.sync_copy(idx_hbm.at[wid], idx_v)

    def gcp(ch, slot):
        return pltpu.make_async_copy(
            table_hbm.at[idx_v.at[ch]], bufs[slot], gsem[slot])

    def scp(ch, slot):
        return pltpu.make_async_copy(
            bufs[slot], out_hbm.at[pl.ds(base + ch * _C, _C)], ssem[slot])

    def step(h, u):
        b2 = (u + _PREF) % _NSLOT
        if not isinstance(h, int) or h - _DRAIN >= 0:
            scp(h - _DRAIN, b2).wait()
        if not isinstance(h, int) or h + _PREF < _NCHUNK:
            gcp(h + _PREF, b2).start()
        gcp(h, u).wait()
        scp(h, u).start()

    for ch in range(_PREF):
        gcp(ch, ch).start()
    for h in range(_NSLOT):
        step(h, h % _NSLOT)

    def body(j, carry):
        h0 = j * _NSLOT
        for u in range(_NSLOT):
            step(h0 + u, u)
        return carry

    lax.fori_loop(1, _NCHUNK // _NSLOT - 1, body, 0)

    for h in range(_NCHUNK - _NSLOT, _NCHUNK):
        step(h, h % _NSLOT)
    for h in range(_NCHUNK - _DRAIN, _NCHUNK):
        scp(h, h % _NSLOT).wait()


def kernel(position_ids, pe):
    idx = position_ids.astype(jnp.int32).reshape(_NW, _NCHUNK, _C)
    out = _gather(idx, pe)
    return out.reshape(_B, _S, _D)


# speedup vs baseline: 1.0065x; 1.0065x over previous
import functools

import jax
import jax.numpy as jnp
from jax import lax
from jax.experimental import pallas as pl
from jax.experimental.pallas import tpu as pltpu
from jax.experimental.pallas import tpu_sc as plsc

_B, _S, _D, _V = 4, 8192, 1024, 8192
_NB = _B * _S
_NC, _NS = 2, 16
_NW = _NC * _NS
_BPW = _NB // _NW
_C = 16
_NCHUNK = _BPW // _C
_NSLOT = 4
_PREF = 3
_DRAIN = _NSLOT - _PREF

_mesh = plsc.VectorSubcoreMesh(core_axis_name="c", subcore_axis_name="s")


@functools.partial(
    pl.kernel,
    mesh=_mesh,
    out_type=jax.ShapeDtypeStruct((_NB, _D), jnp.float32),
    scratch_types=(
        [pltpu.VMEM((_BPW,), jnp.int32)]
        + [pltpu.VMEM((_C, _D), jnp.float32)] * _NSLOT
        + [pltpu.SemaphoreType.DMA] * (2 * _NSLOT)
    ),
)
def _gather(idx_hbm, table_hbm, out_hbm, idx_v, *bufs_and_sems):
    bufs = bufs_and_sems[:_NSLOT]
    gsem = bufs_and_sems[_NSLOT:2 * _NSLOT]
    ssem = bufs_and_sems[2 * _NSLOT:]

    wid = lax.axis_index("s") * _NC + lax.axis_index("c")
    base = wid * _BPW
    pltpu.sync_copy(
        idx_hbm.at[wid // (_S // _BPW), pl.ds((wid % (_S // _BPW)) * _BPW, _BPW)],
        idx_v)

    def gcp(ch, slot):
        return pltpu.make_async_copy(
            table_hbm.at[idx_v.at[pl.ds(ch * _C, _C)]], bufs[slot], gsem[slot])

    def scp(ch, slot):
        return pltpu.make_async_copy(
            bufs[slot], out_hbm.at[pl.ds(base + ch * _C, _C)], ssem[slot])

    def step(h, u):
        b2 = (u + _PREF) % _NSLOT
        if not isinstance(h, int) or h - _DRAIN >= 0:
            scp(h - _DRAIN, b2).wait()
        if not isinstance(h, int) or h + _PREF < _NCHUNK:
            gcp(h + _PREF, b2).start()
        gcp(h, u).wait()
        scp(h, u).start()

    for ch in range(_PREF):
        gcp(ch, ch).start()
    for h in range(_NSLOT):
        step(h, h % _NSLOT)

    def body(j, carry):
        h0 = j * _NSLOT
        for u in range(_NSLOT):
            step(h0 + u, u)
        return carry

    lax.fori_loop(1, _NCHUNK // _NSLOT - 1, body, 0)

    for h in range(_NCHUNK - _NSLOT, _NCHUNK):
        step(h, h % _NSLOT)
    for h in range(_NCHUNK - _DRAIN, _NCHUNK):
        scp(h, h % _NSLOT).wait()


def kernel(position_ids, pe):
    idx = position_ids.astype(jnp.int32)
    out = _gather(idx, pe)
    return out.reshape(_B, _S, _D)
